# baseline (speedup 1.0000x reference)
import functools

import jax
import jax.numpy as jnp
from jax import lax
from jax.experimental import pallas as pl
from jax.experimental.pallas import tpu as pltpu
from jax.experimental.pallas import tpu_sc as plsc

_CHUNK = 128


@functools.cache
def _build(B, D, dtype):
    info = plsc.get_sparse_core_info()
    NC, NS = info.num_cores, info.num_subcores
    NW = NC * NS
    b_per_w = B // NW
    n_chunks = b_per_w // _CHUNK
    mesh = plsc.VectorSubcoreMesh(core_axis_name="c", subcore_axis_name="s")

    @functools.partial(
        pl.kernel,
        mesh=mesh,
        compiler_params=pltpu.CompilerParams(use_tc_tiling_on_sc=False),
        out_type=(
            jax.ShapeDtypeStruct((B, D), dtype),
            jax.ShapeDtypeStruct((B, D), dtype),
        ),
        scratch_types=[
            pltpu.VMEM((b_per_w,), jnp.int32),
            pltpu.VMEM((b_per_w, D), dtype),
            pltpu.VMEM((b_per_w,), jnp.int32),
            pltpu.VMEM((b_per_w, D), dtype),
            pltpu.SemaphoreType.DMA,
            pltpu.SemaphoreType.DMA,
        ],
    )
    def k(users_hbm, items_hbm, tab_hbm, uout_hbm, iout_hbm,
          uidx_v, urows_v, iidx_v, irows_v, usem, isem):
        wid = lax.axis_index("s") * NC + lax.axis_index("c")
        base = wid * b_per_w
        pltpu.sync_copy(users_hbm.at[pl.ds(base, b_per_w)], uidx_v)
        pltpu.sync_copy(items_hbm.at[pl.ds(base, b_per_w)], iidx_v)
        ucps = []
        icps = []
        for j in range(n_chunks):
            s = pl.ds(j * _CHUNK, _CHUNK)
            ucps.append(pltpu.async_copy(
                tab_hbm.at[uidx_v.at[s]], urows_v.at[s], usem))
            icps.append(pltpu.async_copy(
                tab_hbm.at[iidx_v.at[s]], irows_v.at[s], isem))
        for cp in ucps:
            cp.wait()
        pltpu.sync_copy(urows_v, uout_hbm.at[pl.ds(base, b_per_w)])
        for cp in icps:
            cp.wait()
        pltpu.sync_copy(irows_v, iout_hbm.at[pl.ds(base, b_per_w)])

    return k


def kernel(users, items, user_table, item_table):
    B = users.shape[0]
    V, D = user_table.shape
    table = jnp.concatenate([user_table, item_table], axis=0)
    k = _build(B, D, user_table.dtype)
    out = k(users.astype(jnp.int32), items.astype(jnp.int32) + V, table)
    return (out[0], out[1])


# speedup vs baseline: 1.5041x; 1.5041x over previous
import functools

import jax
import jax.numpy as jnp
from jax import lax
from jax.experimental import pallas as pl
from jax.experimental.pallas import tpu as pltpu
from jax.experimental.pallas import tpu_sc as plsc

_CHUNK = 128


@functools.cache
def _build(B, D, dtype):
    info = plsc.get_sparse_core_info()
    NC, NS = info.num_cores, info.num_subcores
    NW = NC * NS
    b_per_w = B // NW
    n_chunks = b_per_w // _CHUNK
    mesh = plsc.VectorSubcoreMesh(core_axis_name="c", subcore_axis_name="s")

    @functools.partial(
        pl.kernel,
        mesh=mesh,
        compiler_params=pltpu.CompilerParams(use_tc_tiling_on_sc=False),
        out_type=jax.ShapeDtypeStruct((B, D), dtype),
        scratch_types=[
            pltpu.VMEM((b_per_w,), jnp.int32),
            pltpu.VMEM((b_per_w, D), dtype),
            pltpu.SemaphoreType.DMA,
        ],
    )
    def k(idx_hbm, tab_hbm, out_hbm, idx_v, rows_v, sem):
        wid = lax.axis_index("s") * NC + lax.axis_index("c")
        base = wid * b_per_w
        pltpu.sync_copy(idx_hbm.at[pl.ds(base, b_per_w)], idx_v)
        cps = []
        for j in range(n_chunks):
            s = pl.ds(j * _CHUNK, _CHUNK)
            cps.append(pltpu.async_copy(
                tab_hbm.at[idx_v.at[s]], rows_v.at[s], sem))
        for cp in cps:
            cp.wait()
        pltpu.sync_copy(rows_v, out_hbm.at[pl.ds(base, b_per_w)])

    return k


def kernel(users, items, user_table, item_table):
    B = users.shape[0]
    D = user_table.shape[1]
    k = _build(B, D, user_table.dtype)
    users_embs = k(users.astype(jnp.int32), user_table)
    items_embs = k(items.astype(jnp.int32), item_table)
    return (users_embs, items_embs)


# speedup vs baseline: 2.1112x; 1.4036x over previous
import functools

import jax
import jax.numpy as jnp
from jax import lax
from jax.experimental import pallas as pl
from jax.experimental.pallas import tpu as pltpu
from jax.experimental.pallas import tpu_sc as plsc

_L = 16


@functools.cache
def _build(B, D, dtype):
    info = plsc.get_sparse_core_info()
    NC, NS = info.num_cores, info.num_subcores
    NW = NC * NS
    b_per_w = B // NW
    half = b_per_w // 2
    mesh = plsc.VectorSubcoreMesh(core_axis_name="c", subcore_axis_name="s")

    @functools.partial(
        pl.kernel,
        mesh=mesh,
        out_type=(
            jax.ShapeDtypeStruct((B, D), dtype),
            jax.ShapeDtypeStruct((B, D), dtype),
        ),
        scratch_types=[
            pltpu.VMEM((b_per_w,), jnp.int32),
            pltpu.VMEM((b_per_w,), jnp.int32),
            pltpu.VMEM((half, D), dtype),
            pltpu.VMEM((half, D), dtype),
            pltpu.SemaphoreType.DMA,
            pltpu.SemaphoreType.DMA,
        ],
    )
    def k(users_hbm, items_hbm, utab_hbm, itab_hbm, uout_hbm, iout_hbm,
          uidx_v, iidx_v, urows_v, irows_v, usem, isem):
        wid = lax.axis_index("s") * NC + lax.axis_index("c")
        base = wid * b_per_w
        pltpu.sync_copy(users_hbm.at[pl.ds(base, b_per_w)], uidx_v)
        pltpu.sync_copy(items_hbm.at[pl.ds(base, b_per_w)], iidx_v)

        lanes = lax.iota(jnp.int32, _L)
        onehot = [(lanes == j).astype(jnp.int32) for j in range(_L)]

        for h in range(2):
            off = h * half

            def fetch(c, _):
                uvec = uidx_v[pl.ds(off + c * _L, _L)]
                ivec = iidx_v[pl.ds(off + c * _L, _L)]
                for j in range(_L):
                    i = c * _L + j
                    r = uvec[j]
                    pltpu.async_copy(utab_hbm.at[pl.ds(r, 1)],
                                     urows_v.at[pl.ds(i, 1)], usem)
                    q = ivec[j]
                    pltpu.async_copy(itab_hbm.at[pl.ds(q, 1)],
                                     irows_v.at[pl.ds(i, 1)], isem)
                return ()

            lax.fori_loop(0, half // _L, fetch, (), unroll=False)
            pltpu.make_async_copy(utab_hbm.at[pl.ds(0, half)], urows_v,
                                  usem).wait()
            pltpu.sync_copy(urows_v, uout_hbm.at[pl.ds(base + off, half)])
            pltpu.make_async_copy(itab_hbm.at[pl.ds(0, half)], irows_v,
                                  isem).wait()
            pltpu.sync_copy(irows_v, iout_hbm.at[pl.ds(base + off, half)])

    return k


def kernel(users, items, user_table, item_table):
    B = users.shape[0]
    D = user_table.shape[1]
    k = _build(B, D, user_table.dtype)
    out = k(users.astype(jnp.int32), items.astype(jnp.int32),
            user_table, item_table)
    return (out[0], out[1])


# speedup vs baseline: 2.2391x; 1.0606x over previous
import functools

import jax
import jax.numpy as jnp
from jax import lax
from jax.experimental import pallas as pl
from jax.experimental.pallas import tpu as pltpu
from jax.experimental.pallas import tpu_sc as plsc

_L = 16


@functools.cache
def _build(B, D, dtype):
    info = plsc.get_sparse_core_info()
    NC, NS = info.num_cores, info.num_subcores
    NW = NC * NS
    b_per_w = B // NW
    half = b_per_w // 2
    mesh = plsc.VectorSubcoreMesh(core_axis_name="c", subcore_axis_name="s")

    @functools.partial(
        pl.kernel,
        mesh=mesh,
        out_type=jax.ShapeDtypeStruct((B, D), dtype),
        scratch_types=[
            pltpu.VMEM((b_per_w,), jnp.int32),
            pltpu.VMEM((half, D), dtype),
            pltpu.SemaphoreType.DMA,
        ],
    )
    def k(idx_hbm, tab_hbm, out_hbm, idx_v, rows_v, sem):
        wid = lax.axis_index("s") * NC + lax.axis_index("c")
        base = wid * b_per_w
        pltpu.sync_copy(idx_hbm.at[pl.ds(base, b_per_w)], idx_v)

        for h in range(2):
            off = h * half

            def fetch(c, _):
                vec = idx_v[pl.ds(off + c * _L, _L)]
                for j in range(_L):
                    i = c * _L + j
                    r = vec[j]
                    pltpu.async_copy(tab_hbm.at[pl.ds(r, 1)],
                                     rows_v.at[pl.ds(i, 1)], sem)
                return ()

            lax.fori_loop(0, half // _L, fetch, (), unroll=False)
            pltpu.make_async_copy(tab_hbm.at[pl.ds(0, half)], rows_v,
                                  sem).wait()
            pltpu.sync_copy(rows_v, out_hbm.at[pl.ds(base + off, half)])

    return k


def kernel(users, items, user_table, item_table):
    B = users.shape[0]
    D = user_table.shape[1]
    k = _build(B, D, user_table.dtype)
    users_embs = k(users.astype(jnp.int32), user_table)
    items_embs = k(items.astype(jnp.int32), item_table)
    return (users_embs, items_embs)
